# baseline (speedup 1.0000x reference)
import jax
import jax.numpy as jnp
from jax import lax
from jax.experimental import pallas as pl
from jax.experimental.pallas import tpu as pltpu
from jax.experimental.pallas import tpu_sc as plsc

N = 10000
E = 320000
D = 128
NC = 2
NS = 16
NW = NC * NS
EPT = E // NW
C = 80
NCH = EPT // C
RPT = N // NS
NG = N // 8
SLOPE = 0.2


BR = 2000


def _proj_body(feats_ref, wsrc_ref, wdst_ref, hs_ref, hd_ref):
    f = feats_ref[...]
    hs_ref[...] = jnp.dot(f, wsrc_ref[...], preferred_element_type=jnp.float32)
    hd_ref[...] = jnp.dot(f, wdst_ref[...], preferred_element_type=jnp.float32)


_proj = pl.pallas_call(
    _proj_body,
    grid=(N // BR,),
    in_specs=[
        pl.BlockSpec((BR, D), lambda i: (i, 0)),
        pl.BlockSpec((D, D), lambda i: (0, 0)),
        pl.BlockSpec((D, D), lambda i: (0, 0)),
    ],
    out_specs=[
        pl.BlockSpec((BR, D), lambda i: (i, 0)),
        pl.BlockSpec((BR, D), lambda i: (i, 0)),
    ],
    out_shape=[jax.ShapeDtypeStruct((N, D), jnp.float32)] * 2,
)


def _edge_body(hsrc_hbm, hdst_hbm, src_hbm, dst_hbm, attn_hbm,
               accum_hbm, denom_hbm,
               sidx0, sidx1, didx0, didx1, hs0, hs1, hd0, hd1,
               sidx_st, didx_st, p80_0, p80_1,
               attn_v, s_v, zden_v, accum_sh, den_sh,
               si, di, gs0, gs1, gd0, gd1, scm0, scm1, scd0, scd1):
    SIDX = [sidx0, sidx1]
    DIDX = [didx0, didx1]
    HS = [hs0, hs1]
    HD = [hd0, hd1]
    P80 = [p80_0, p80_1]
    GS = [gs0, gs1]
    GD = [gd0, gd1]
    SCM = [scm0, scm1]
    SCD = [scd0, scd1]

    cid = lax.axis_index("c")
    sid = lax.axis_index("s")
    ebase = (cid * NS + sid) * EPT

    pltpu.sync_copy(attn_hbm, attn_v)

    zero16 = jnp.zeros((16,), jnp.float32)

    def _zmsg(r, carry):
        for d in range(8):
            hs0[r, pl.ds(d * 16, 16)] = zero16
        return carry

    lax.fori_loop(0, C, _zmsg, 0)
    for i in range(40):
        zden_v[pl.ds(i * 16, 16)] = zero16
    for k in range(7):
        pltpu.sync_copy(hs0, accum_sh.at[pl.ds(sid * RPT + k * 80, 80)])
    pltpu.sync_copy(hs0.at[pl.ds(0, 65)],
                    accum_sh.at[pl.ds(sid * RPT + 560, 65)])
    pltpu.sync_copy(zden_v.at[pl.ds(0, 624)], den_sh.at[pl.ds(sid * 624, 624)])

    @pl.when(sid == 0)
    def _():
        pltpu.sync_copy(zden_v.at[pl.ds(0, 16)], den_sh.at[pl.ds(16 * 624, 16)])
    plsc.subcore_barrier()

    a_regs = [attn_v[pl.ds(d * 16, 16)] for d in range(8)]
    idx_base = lax.iota(jnp.int32, 16) * 16

    def _compute(b, q0, q1):
        hs_v, hd_v, p80_v = HS[b], HD[b], P80[b]

        def _grp(q, carry2):
            e0 = q * 16
            for j in range(16):
                e = e0 + j
                acc = zero16
                for d in range(8):
                    x = hs_v[e, pl.ds(d * 16, 16)] + hd_v[e, pl.ds(d * 16, 16)]
                    t = jnp.maximum(x, SLOPE * x)
                    acc = acc + t * a_regs[d]
                s_v[pl.ds(j * 16, 16)] = acc
            tot = zero16
            for c in range(16):
                tot = tot + plsc.load_gather(s_v, [idx_base + c])
            p16 = jnp.exp(tot)
            p80_v[pl.ds(e0, 16)] = p16
            for j in range(16):
                e = e0 + j
                pj = p16[j]
                for d in range(8):
                    hs_v[e, pl.ds(d * 16, 16)] = pj * hs_v[e, pl.ds(d * 16, 16)]
            return carry2

        lax.fori_loop(q0, q1, _grp, 0)

    pltpu.sync_copy(src_hbm.at[pl.ds(ebase, C)], sidx0)
    pltpu.sync_copy(dst_hbm.at[pl.ds(ebase, C)], didx0)
    pltpu.async_copy(src_hbm.at[pl.ds(ebase + C, C)], sidx_st, si)
    pltpu.async_copy(dst_hbm.at[pl.ds(ebase + C, C)], didx_st, di)
    pltpu.async_copy(hsrc_hbm.at[sidx0], hs0, gs0)
    pltpu.async_copy(hdst_hbm.at[didx0], hd0, gd0)

    def _iter(g, b, first):
        nb = 1 - b
        pltpu.make_async_copy(hsrc_hbm.at[SIDX[b]], HS[b], GS[b]).wait()
        pltpu.make_async_copy(hdst_hbm.at[DIDX[b]], HD[b], GD[b]).wait()
        _compute(b, 0, 2)
        if not first:
            pltpu.make_async_copy(HS[nb], accum_sh.at[DIDX[nb]],
                                  SCM[nb]).wait()
            pltpu.make_async_copy(P80[nb], den_sh.at[DIDX[nb]],
                                  SCD[nb]).wait()

        @pl.when(g < NCH - 1)
        def _():
            pltpu.make_async_copy(src_hbm.at[pl.ds(ebase + (g + 1) * C, C)],
                                  sidx_st, si).wait()
            pltpu.make_async_copy(dst_hbm.at[pl.ds(ebase + (g + 1) * C, C)],
                                  didx_st, di).wait()
            for t in range(C // 16):
                SIDX[nb][pl.ds(t * 16, 16)] = sidx_st[pl.ds(t * 16, 16)]
                DIDX[nb][pl.ds(t * 16, 16)] = didx_st[pl.ds(t * 16, 16)]
            pltpu.async_copy(hsrc_hbm.at[SIDX[nb]], HS[nb], GS[nb])
            pltpu.async_copy(hdst_hbm.at[DIDX[nb]], HD[nb], GD[nb])

        @pl.when(g < NCH - 2)
        def _():
            pltpu.async_copy(src_hbm.at[pl.ds(ebase + (g + 2) * C, C)],
                             sidx_st, si)
            pltpu.async_copy(dst_hbm.at[pl.ds(ebase + (g + 2) * C, C)],
                             didx_st, di)

        _compute(b, 2, C // 16)
        pltpu.async_copy(HS[b], accum_sh.at[DIDX[b]], SCM[b], add=True)
        pltpu.async_copy(P80[b], den_sh.at[DIDX[b]], SCD[b], add=True)

    _iter(0, 0, True)

    def _pair(i, carry):
        g = 2 * i + 1
        _iter(g, 1, False)
        _iter(g + 1, 0, False)
        return carry

    lax.fori_loop(0, (NCH - 1) // 2, _pair, 0)
    pltpu.make_async_copy(HS[0], accum_sh.at[DIDX[0]], SCM[0]).wait()
    pltpu.make_async_copy(P80[0], den_sh.at[DIDX[0]], SCD[0]).wait()

    plsc.subcore_barrier()
    ngrp = jnp.where(sid < NG - 78 * NS, 79, 78)

    def _out(k, carry):
        grp = k * NS + sid
        pltpu.sync_copy(accum_sh.at[pl.ds(grp * 8, 8)],
                        accum_hbm.at[cid, pl.ds(grp * 8, 8)])
        return carry

    lax.fori_loop(0, ngrp, _out, 0)

    pltpu.sync_copy(den_sh.at[pl.ds(sid * 624, 624)], zden_v.at[pl.ds(0, 624)])
    pltpu.sync_copy(zden_v.at[pl.ds(0, 624)],
                    denom_hbm.at[pl.ds(cid * N + sid * 624, 624)])

    @pl.when(sid == 0)
    def _():
        pltpu.sync_copy(den_sh.at[pl.ds(16 * 624, 16)], zden_v.at[pl.ds(0, 16)])
        pltpu.sync_copy(zden_v.at[pl.ds(0, 16)],
                        denom_hbm.at[pl.ds(cid * N + 16 * 624, 16)])


_edge_kernel = pl.kernel(
    _edge_body,
    out_type=(jax.ShapeDtypeStruct((NC, N, D), jnp.float32),
              jax.ShapeDtypeStruct((NC * N,), jnp.float32)),
    mesh=plsc.VectorSubcoreMesh(core_axis_name="c", subcore_axis_name="s",
                                num_cores=NC, num_subcores=NS),
    scratch_types=(
        pltpu.VMEM((C,), jnp.int32),
        pltpu.VMEM((C,), jnp.int32),
        pltpu.VMEM((C,), jnp.int32),
        pltpu.VMEM((C,), jnp.int32),
        pltpu.VMEM((C, D), jnp.float32),
        pltpu.VMEM((C, D), jnp.float32),
        pltpu.VMEM((C, D), jnp.float32),
        pltpu.VMEM((C, D), jnp.float32),
        pltpu.VMEM((C,), jnp.int32),
        pltpu.VMEM((C,), jnp.int32),
        pltpu.VMEM((C,), jnp.float32),
        pltpu.VMEM((C,), jnp.float32),
        pltpu.VMEM((D,), jnp.float32),
        pltpu.VMEM((256,), jnp.float32),
        pltpu.VMEM((640,), jnp.float32),
        pltpu.VMEM_SHARED((N, D), jnp.float32),
        pltpu.VMEM_SHARED((N,), jnp.float32),
        pltpu.SemaphoreType.DMA,
        pltpu.SemaphoreType.DMA,
        pltpu.SemaphoreType.DMA,
        pltpu.SemaphoreType.DMA,
        pltpu.SemaphoreType.DMA,
        pltpu.SemaphoreType.DMA,
        pltpu.SemaphoreType.DMA,
        pltpu.SemaphoreType.DMA,
        pltpu.SemaphoreType.DMA,
        pltpu.SemaphoreType.DMA,
    ),
    compiler_params=pltpu.CompilerParams(needs_layout_passes=False),
)


def _readout_body(feats_ref, acc_ref, den_ref, bias_ref,
                  r0w_ref, r0b_ref, r1w_ref, r1b_ref, out_ref):
    f = feats_ref[...]
    acc = acc_ref[0] + acc_ref[1]
    den = den_ref[0] + den_ref[1]
    h = acc / (den[:, None] + 1e-9) + bias_ref[...]
    f1 = jnp.maximum(h, SLOPE * h)
    out_ref[...] = (jnp.dot(f, r0w_ref[...], preferred_element_type=jnp.float32)
                    + jnp.dot(f1, r1w_ref[...], preferred_element_type=jnp.float32)
                    + r0b_ref[...] + r1b_ref[...])


_readout = pl.pallas_call(
    _readout_body,
    out_shape=jax.ShapeDtypeStruct((N, D), jnp.float32),
)


def kernel(feats, edge_index, W_src, W_dst, attn, bias, R0_W, R0_b, R1_W, R1_b):
    src = edge_index[0].astype(jnp.int32).reshape(E)
    dst = edge_index[1].astype(jnp.int32).reshape(E)
    hs, hd = _proj(feats, W_src, W_dst)
    accum, denom = _edge_kernel(hs, hd, src, dst, attn.reshape(D))
    return _readout(feats, accum, denom.reshape(NC, N), bias.reshape(1, D),
                    R0_W, R0_b.reshape(1, D), R1_W, R1_b.reshape(1, D))


# speedup vs baseline: 1.0945x; 1.0945x over previous
import jax
import jax.numpy as jnp
from jax import lax
from jax.experimental import pallas as pl
from jax.experimental.pallas import tpu as pltpu
from jax.experimental.pallas import tpu_sc as plsc

N = 10000
E = 320000
D = 128
NC = 2
NS = 16
NW = NC * NS
EPT = E // NW
C = 80
NCH = EPT // C
RPT = N // NS
NG = N // 8
SLOPE = 0.2


BR = 2000


def _proj_body(feats_ref, wsrc_ref, wdst_ref, hs_ref, hd_ref):
    f = feats_ref[...]
    hs_ref[...] = jnp.dot(f, wsrc_ref[...], preferred_element_type=jnp.float32)
    hd_ref[...] = jnp.dot(f, wdst_ref[...], preferred_element_type=jnp.float32)


_proj = pl.pallas_call(
    _proj_body,
    grid=(N // BR,),
    in_specs=[
        pl.BlockSpec((BR, D), lambda i: (i, 0)),
        pl.BlockSpec((D, D), lambda i: (0, 0)),
        pl.BlockSpec((D, D), lambda i: (0, 0)),
    ],
    out_specs=[
        pl.BlockSpec((BR, D), lambda i: (i, 0)),
        pl.BlockSpec((BR, D), lambda i: (i, 0)),
    ],
    out_shape=[jax.ShapeDtypeStruct((N, D), jnp.float32)] * 2,
)


def _edge_body(hsrc_hbm, hdst_hbm, src_hbm, dst_hbm, attn_hbm,
               accum_hbm, denom_hbm,
               sidx0, sidx1, didx0, didx1, hs0, hs1, hd0, hd1,
               attn_v, s_v, p80_v, zden_v, accum_sh, den_sh,
               si0, si1, di0, di1, gs0, gs1, gd0, gd1, sem_sc):
    SIDX = [sidx0, sidx1]
    DIDX = [didx0, didx1]
    HS = [hs0, hs1]
    HD = [hd0, hd1]
    SI = [si0, si1]
    DI = [di0, di1]
    GS = [gs0, gs1]
    GD = [gd0, gd1]

    cid = lax.axis_index("c")
    sid = lax.axis_index("s")
    ebase = (cid * NS + sid) * EPT

    pltpu.sync_copy(attn_hbm, attn_v)

    zero16 = jnp.zeros((16,), jnp.float32)

    def _zmsg(r, carry):
        for d in range(8):
            hs0[r, pl.ds(d * 16, 16)] = zero16
        return carry

    lax.fori_loop(0, C, _zmsg, 0)
    for i in range(40):
        zden_v[pl.ds(i * 16, 16)] = zero16
    for k in range(7):
        pltpu.sync_copy(hs0, accum_sh.at[pl.ds(sid * RPT + k * 80, 80)])
    pltpu.sync_copy(hs0.at[pl.ds(0, 65)],
                    accum_sh.at[pl.ds(sid * RPT + 560, 65)])
    pltpu.sync_copy(zden_v.at[pl.ds(0, 624)], den_sh.at[pl.ds(sid * 624, 624)])

    @pl.when(sid == 0)
    def _():
        pltpu.sync_copy(zden_v.at[pl.ds(0, 16)], den_sh.at[pl.ds(16 * 624, 16)])
    plsc.subcore_barrier()

    a_regs = [attn_v[pl.ds(d * 16, 16)] for d in range(8)]
    idx_base = lax.iota(jnp.int32, 16) * 16

    def _compute(b):
        hs_v, hd_v, didx_v = HS[b], HD[b], DIDX[b]

        def _grp(q, carry2):
            e0 = q * 16
            for j in range(16):
                e = e0 + j
                acc = zero16
                for d in range(8):
                    x = hs_v[e, pl.ds(d * 16, 16)] + hd_v[e, pl.ds(d * 16, 16)]
                    t = jnp.maximum(x, SLOPE * x)
                    acc = acc + t * a_regs[d]
                s_v[pl.ds(j * 16, 16)] = acc
            tot = zero16
            for c in range(16):
                tot = tot + plsc.load_gather(s_v, [idx_base + c])
            p16 = jnp.exp(tot)
            p80_v[pl.ds(e0, 16)] = p16
            for j in range(16):
                e = e0 + j
                pj = p16[j]
                for d in range(8):
                    hs_v[e, pl.ds(d * 16, 16)] = pj * hs_v[e, pl.ds(d * 16, 16)]
            return carry2

        lax.fori_loop(0, C // 16, _grp, 0)
        cp_m = pltpu.async_copy(hs_v, accum_sh.at[didx_v], sem_sc, add=True)
        pltpu.sync_copy(p80_v, den_sh.at[didx_v], add=True)
        cp_m.wait()

    pltpu.sync_copy(src_hbm.at[pl.ds(ebase, C)], sidx0)
    pltpu.sync_copy(dst_hbm.at[pl.ds(ebase, C)], didx0)
    pltpu.async_copy(src_hbm.at[pl.ds(ebase + C, C)], sidx1, si1)
    pltpu.async_copy(dst_hbm.at[pl.ds(ebase + C, C)], didx1, di1)
    pltpu.async_copy(hsrc_hbm.at[sidx0], hs0, gs0)
    pltpu.async_copy(hdst_hbm.at[didx0], hd0, gd0)

    def _iter(g, b):
        nb = 1 - b
        pltpu.make_async_copy(src_hbm.at[pl.ds(ebase + (g + 1) * C, C)],
                              SIDX[nb], SI[nb]).wait()
        pltpu.make_async_copy(dst_hbm.at[pl.ds(ebase + (g + 1) * C, C)],
                              DIDX[nb], DI[nb]).wait()
        pltpu.async_copy(hsrc_hbm.at[SIDX[nb]], HS[nb], GS[nb])
        pltpu.async_copy(hdst_hbm.at[DIDX[nb]], HD[nb], GD[nb])
        pltpu.make_async_copy(hsrc_hbm.at[SIDX[b]], HS[b], GS[b]).wait()
        pltpu.make_async_copy(hdst_hbm.at[DIDX[b]], HD[b], GD[b]).wait()
        _compute(b)

        @pl.when(g < NCH - 2)
        def _():
            pltpu.async_copy(src_hbm.at[pl.ds(ebase + (g + 2) * C, C)],
                             SIDX[b], SI[b])
            pltpu.async_copy(dst_hbm.at[pl.ds(ebase + (g + 2) * C, C)],
                             DIDX[b], DI[b])

    def _pair(i, carry):
        g = 2 * i
        _iter(g, 0)
        _iter(g + 1, 1)
        return carry

    lax.fori_loop(0, (NCH - 1) // 2, _pair, 0)
    pltpu.make_async_copy(hsrc_hbm.at[sidx0], hs0, gs0).wait()
    pltpu.make_async_copy(hdst_hbm.at[didx0], hd0, gd0).wait()
    _compute(0)

    plsc.subcore_barrier()
    ngrp = jnp.where(sid < NG - 78 * NS, 79, 78)

    def _out(k, carry):
        grp = k * NS + sid
        pltpu.sync_copy(accum_sh.at[pl.ds(grp * 8, 8)],
                        accum_hbm.at[cid, pl.ds(grp * 8, 8)])
        return carry

    lax.fori_loop(0, ngrp, _out, 0)

    pltpu.sync_copy(den_sh.at[pl.ds(sid * 624, 624)], zden_v.at[pl.ds(0, 624)])
    pltpu.sync_copy(zden_v.at[pl.ds(0, 624)],
                    denom_hbm.at[pl.ds(cid * N + sid * 624, 624)])

    @pl.when(sid == 0)
    def _():
        pltpu.sync_copy(den_sh.at[pl.ds(16 * 624, 16)], zden_v.at[pl.ds(0, 16)])
        pltpu.sync_copy(zden_v.at[pl.ds(0, 16)],
                        denom_hbm.at[pl.ds(cid * N + 16 * 624, 16)])


_edge_kernel = pl.kernel(
    _edge_body,
    out_type=(jax.ShapeDtypeStruct((NC, N, D), jnp.float32),
              jax.ShapeDtypeStruct((NC * N,), jnp.float32)),
    mesh=plsc.VectorSubcoreMesh(core_axis_name="c", subcore_axis_name="s",
                                num_cores=NC, num_subcores=NS),
    scratch_types=(
        pltpu.VMEM((C,), jnp.int32),
        pltpu.VMEM((C,), jnp.int32),
        pltpu.VMEM((C,), jnp.int32),
        pltpu.VMEM((C,), jnp.int32),
        pltpu.VMEM((C, D), jnp.float32),
        pltpu.VMEM((C, D), jnp.float32),
        pltpu.VMEM((C, D), jnp.float32),
        pltpu.VMEM((C, D), jnp.float32),
        pltpu.VMEM((D,), jnp.float32),
        pltpu.VMEM((256,), jnp.float32),
        pltpu.VMEM((C,), jnp.float32),
        pltpu.VMEM((640,), jnp.float32),
        pltpu.VMEM_SHARED((N, D), jnp.float32),
        pltpu.VMEM_SHARED((N,), jnp.float32),
        pltpu.SemaphoreType.DMA,
        pltpu.SemaphoreType.DMA,
        pltpu.SemaphoreType.DMA,
        pltpu.SemaphoreType.DMA,
        pltpu.SemaphoreType.DMA,
        pltpu.SemaphoreType.DMA,
        pltpu.SemaphoreType.DMA,
        pltpu.SemaphoreType.DMA,
        pltpu.SemaphoreType.DMA,
    ),
    compiler_params=pltpu.CompilerParams(needs_layout_passes=False),
)


def _readout_body(feats_ref, acc_ref, den_ref, bias_ref,
                  r0w_ref, r0b_ref, r1w_ref, r1b_ref, out_ref):
    f = feats_ref[...]
    acc = acc_ref[0] + acc_ref[1]
    den = den_ref[0] + den_ref[1]
    h = acc / (den[:, None] + 1e-9) + bias_ref[...]
    f1 = jnp.maximum(h, SLOPE * h)
    out_ref[...] = (jnp.dot(f, r0w_ref[...], preferred_element_type=jnp.float32)
                    + jnp.dot(f1, r1w_ref[...], preferred_element_type=jnp.float32)
                    + r0b_ref[...] + r1b_ref[...])


_readout = pl.pallas_call(
    _readout_body,
    out_shape=jax.ShapeDtypeStruct((N, D), jnp.float32),
)


def kernel(feats, edge_index, W_src, W_dst, attn, bias, R0_W, R0_b, R1_W, R1_b):
    src = edge_index[0].astype(jnp.int32).reshape(E)
    dst = edge_index[1].astype(jnp.int32).reshape(E)
    hs, hd = _proj(feats, W_src, W_dst)
    accum, denom = _edge_kernel(hs, hd, src, dst, attn.reshape(D))
    return _readout(feats, accum, denom.reshape(NC, N), bias.reshape(1, D),
                    R0_W, R0_b.reshape(1, D), R1_W, R1_b.reshape(1, D))
